# Initial kernel scaffold; baseline (speedup 1.0000x reference)
import jax
import jax.numpy as jnp
from jax.experimental import pallas as pl


def kernel(final_boxes, final_scores, final_classes, descriptors, places_db):
    raise NotImplementedError("write your pallas kernel here")


# speedup vs baseline: 1.2998x; 1.2998x over previous
import jax
import jax.numpy as jnp
from jax.experimental import pallas as pl
from jax.experimental.pallas import tpu as pltpu

_TOPK = 5
_MIN_SIM = 0.8
_MIN_VOTES = 0.0
_NQ = 64
_DIM = 64
_NDB = 100000
_CHUNK = 4000
_NCHUNK = _NDB // _CHUNK
_PAD = 8
_NEG = float("-inf")


def _first_argmax(vals):
    w = vals.shape[1]
    it = jax.lax.broadcasted_iota(jnp.int32, vals.shape, 1)
    m = jnp.max(vals, axis=1, keepdims=True)
    p = jnp.min(jnp.where(vals == m, it, w), axis=1, keepdims=True)
    return m, it == p


def _body(desc_ref, db_ref, scores_ref, classes_ref, rv_ref, ri_ref):
    pid = pl.program_id(0)
    nprog = pl.num_programs(0)

    @pl.when(pid == 0)
    def _init():
        rv_ref[...] = jnp.full((_NQ, _PAD), _NEG, jnp.float32)
        ri_ref[...] = jnp.zeros((_NQ, _PAD), jnp.float32)

    x = db_ref[:, :_DIM]
    ids_col = db_ref[:, _DIM:_DIM + 1]
    q_col = jnp.floor(ids_col * 0.125)
    r_col = ids_col - 8.0 * q_col
    sims = jax.lax.dot_general(
        desc_ref[...], x, (((1,), (1,)), ((), ())),
        preferred_element_type=jnp.float32)

    pad_v = jnp.full((_NQ, _PAD - _TOPK), _NEG, jnp.float32)
    pad_z = jnp.zeros((_NQ, _PAD - _TOPK), jnp.float32)
    cv, ci = [], []
    vals = sims
    for _ in range(_TOPK):
        m, oh = _first_argmax(vals)
        ohf = oh.astype(jnp.float32)
        qsel = jax.lax.dot_general(
            ohf, q_col, (((1,), (0,)), ((), ())),
            preferred_element_type=jnp.float32)
        rsel = jax.lax.dot_general(
            ohf, r_col, (((1,), (0,)), ((), ())),
            preferred_element_type=jnp.float32)
        idsel = 8.0 * qsel + rsel
        cv.append(m)
        ci.append(idsel)
        vals = jnp.where(oh, _NEG, vals)
    cand_v = jnp.concatenate(cv + [pad_v], 1)
    cand_i = jnp.concatenate(ci + [pad_z], 1)

    mv = jnp.concatenate([rv_ref[...], cand_v], 1)
    mi = jnp.concatenate([ri_ref[...], cand_i], 1)
    nv, ni = [], []
    for _ in range(_TOPK):
        m, oh = _first_argmax(mv)
        nv.append(m)
        ni.append(jnp.sum(jnp.where(oh, mi, 0.0), axis=1, keepdims=True))
        mv = jnp.where(oh, _NEG, mv)
    rv = jnp.concatenate(nv + [pad_v], 1)
    ri = jnp.concatenate(ni + [pad_z], 1)
    rv_ref[...] = rv
    ri_ref[...] = ri

    @pl.when(pid == nprog - 1)
    def _finish():
        vals8, ids8 = rv, ri
        mask8 = vals8 >= _MIN_SIM
        counts = jnp.zeros((_NQ, _PAD), jnp.float32)
        for l in range(_TOPK):
            eq = (ids8 == ids8[:, l:l + 1]).astype(jnp.float32)
            counts = counts + jnp.where(mask8[:, l:l + 1], eq, 0.0)
        score = counts * 1e6 - ids8
        score = jnp.where(mask8, score, _NEG)
        m, oh = _first_argmax(score)
        maj = jnp.sum(jnp.where(oh, ids8, 0.0), axis=1, keepdims=True)
        majc = jnp.sum(jnp.where(oh, counts, 0.0), axis=1, keepdims=True)
        numv = jnp.sum(mask8.astype(jnp.float32), axis=1, keepdims=True)
        anyv = numv > 0
        ratio = majc / jnp.maximum(numv, 1.0)
        accepted = jnp.logical_and(anyv, ratio >= _MIN_VOTES)
        match = ids8 == maj
        s = jnp.max(jnp.where(match, vals8, _NEG), axis=1, keepdims=True)
        scores_ref[...] = jnp.where(accepted, s, 0.0)
        classes_ref[...] = jnp.where(accepted, maj.astype(jnp.int32), -1)


@jax.jit
def _run(descriptors, places_db):
    scores, classes = pl.pallas_call(
        _body,
        grid=(_NCHUNK,),
        in_specs=[
            pl.BlockSpec((_NQ, _DIM), lambda i: (0, 0)),
            pl.BlockSpec((_CHUNK, _DIM + 1), lambda i: (i, 0)),
        ],
        out_specs=[
            pl.BlockSpec((_NQ, 1), lambda i: (0, 0)),
            pl.BlockSpec((_NQ, 1), lambda i: (0, 0)),
        ],
        out_shape=[
            jax.ShapeDtypeStruct((_NQ, 1), jnp.float32),
            jax.ShapeDtypeStruct((_NQ, 1), jnp.int32),
        ],
        scratch_shapes=[
            pltpu.VMEM((_NQ, _PAD), jnp.float32),
            pltpu.VMEM((_NQ, _PAD), jnp.float32),
        ],
        compiler_params=pltpu.CompilerParams(
            dimension_semantics=("arbitrary",),
        ),
    )(descriptors, places_db)
    return scores[:, 0], classes[:, 0]


def kernel(final_boxes, final_scores, final_classes, descriptors, places_db):
    scores, classes = _run(descriptors, places_db)
    return final_boxes, scores, classes


# speedup vs baseline: 1.6927x; 1.3022x over previous
import jax
import jax.numpy as jnp
from jax.experimental import pallas as pl
from jax.experimental.pallas import tpu as pltpu

_TOPK = 5
_MIN_SIM = 0.8
_MIN_VOTES = 0.0
_NQ = 64
_DIM = 64
_NDB = 100000
_CHUNK = 4096
_NCHUNK = (_NDB + _CHUNK - 1) // _CHUNK
_LANES = 128
_G = _CHUNK // _LANES
_PAD = 8
_NEG = float("-inf")


def _first_argmax(vals):
    w = vals.shape[1]
    it = jax.lax.broadcasted_iota(jnp.int32, vals.shape, 1)
    m = jnp.max(vals, axis=1, keepdims=True)
    p = jnp.min(jnp.where(vals == m, it, w), axis=1, keepdims=True)
    return m, it == p


def _body(desc_ref, db_ref, scores_ref, classes_ref, rv_ref, ri_ref,
          cv_ref, ci_ref):
    pid = pl.program_id(0)
    nprog = pl.num_programs(0)

    @pl.when(pid == 0)
    def _init():
        rv_ref[...] = jnp.full((_NQ, _PAD), _NEG, jnp.float32)
        ri_ref[...] = jnp.zeros((_NQ, _PAD), jnp.float32)

    x = db_ref[:, :_DIM]
    ids_col = db_ref[:, _DIM:_DIM + 1]
    sims = jax.lax.dot_general(
        desc_ref[...], x, (((1,), (1,)), ((), ())),
        preferred_element_type=jnp.float32)
    ids_row = ids_col.reshape(1, _CHUNK)

    t1 = jnp.full((_NQ, _LANES), _NEG, jnp.float32)
    i1 = jnp.zeros((_NQ, _LANES), jnp.float32)
    t2 = jnp.full((_NQ, _LANES), _NEG, jnp.float32)
    i2 = jnp.zeros((_NQ, _LANES), jnp.float32)
    t3 = jnp.full((_NQ, _LANES), _NEG, jnp.float32)
    for j in range(_G):
        v = sims[:, j * _LANES:(j + 1) * _LANES]
        idr = ids_row[:, j * _LANES:(j + 1) * _LANES]
        gt = v > t1
        dispv = jnp.where(gt, t1, v)
        dispi = jnp.where(gt, i1, idr)
        t1 = jnp.where(gt, v, t1)
        i1 = jnp.where(gt, idr, i1)
        gt2 = dispv > t2
        disp2 = jnp.where(gt2, t2, dispv)
        t2 = jnp.where(gt2, dispv, t2)
        i2 = jnp.where(gt2, dispi, i2)
        t3 = jnp.maximum(t3, disp2)

    cand_v = jnp.concatenate([t1, t2], 1)
    cand_i = jnp.concatenate([i1, i2], 1)
    pad_v = jnp.full((_NQ, _PAD - _TOPK), _NEG, jnp.float32)
    pad_z = jnp.zeros((_NQ, _PAD - _TOPK), jnp.float32)
    cvs, cis = [], []
    vals = cand_v
    for _ in range(_TOPK):
        m = jnp.max(vals, axis=1, keepdims=True)
        oh = vals == m
        cvs.append(m)
        cis.append(jnp.sum(jnp.where(oh, cand_i, 0.0), axis=1, keepdims=True))
        vals = jnp.where(oh, _NEG, vals)
    t5 = cvs[-1]
    bad = jnp.logical_or(jnp.any(t3 >= t5), pid == nprog - 1)
    cv_ref[...] = jnp.concatenate(cvs + [pad_v], 1)
    ci_ref[...] = jnp.concatenate(cis + [pad_z], 1)

    @pl.when(bad)
    def _exact():
        limit = _NDB - pid * _CHUNK
        it = jax.lax.broadcasted_iota(jnp.int32, (_NQ, _CHUNK), 1)
        s2 = jnp.where(it < limit, sims, _NEG)
        rit = jax.lax.broadcasted_iota(jnp.int32, (_CHUNK, 1), 0)
        idm = jnp.where(rit < limit, ids_col, 0.0)
        q_col = jnp.floor(idm * 0.125)
        r_col = idm - 8.0 * q_col
        evs, eis = [], []
        ev = s2
        for _ in range(_TOPK):
            m, oh = _first_argmax(ev)
            ohf = oh.astype(jnp.float32)
            qsel = jax.lax.dot_general(
                ohf, q_col, (((1,), (0,)), ((), ())),
                preferred_element_type=jnp.float32)
            rsel = jax.lax.dot_general(
                ohf, r_col, (((1,), (0,)), ((), ())),
                preferred_element_type=jnp.float32)
            evs.append(m)
            eis.append(8.0 * qsel + rsel)
            ev = jnp.where(oh, _NEG, ev)
        cv_ref[...] = jnp.concatenate(evs + [pad_v], 1)
        ci_ref[...] = jnp.concatenate(eis + [pad_z], 1)

    mv = jnp.concatenate([rv_ref[...], cv_ref[...]], 1)
    mi = jnp.concatenate([ri_ref[...], ci_ref[...]], 1)
    nv, ni = [], []
    for _ in range(_TOPK):
        m, oh = _first_argmax(mv)
        nv.append(m)
        ni.append(jnp.sum(jnp.where(oh, mi, 0.0), axis=1, keepdims=True))
        mv = jnp.where(oh, _NEG, mv)
    rv = jnp.concatenate(nv + [pad_v], 1)
    ri = jnp.concatenate(ni + [pad_z], 1)
    rv_ref[...] = rv
    ri_ref[...] = ri

    @pl.when(pid == nprog - 1)
    def _finish():
        vals8, ids8 = rv, ri
        mask8 = vals8 >= _MIN_SIM
        counts = jnp.zeros((_NQ, _PAD), jnp.float32)
        for l in range(_TOPK):
            eq = (ids8 == ids8[:, l:l + 1]).astype(jnp.float32)
            counts = counts + jnp.where(mask8[:, l:l + 1], eq, 0.0)
        score = counts * 1e6 - ids8
        score = jnp.where(mask8, score, _NEG)
        m, oh = _first_argmax(score)
        maj = jnp.sum(jnp.where(oh, ids8, 0.0), axis=1, keepdims=True)
        majc = jnp.sum(jnp.where(oh, counts, 0.0), axis=1, keepdims=True)
        numv = jnp.sum(mask8.astype(jnp.float32), axis=1, keepdims=True)
        anyv = numv > 0
        ratio = majc / jnp.maximum(numv, 1.0)
        accepted = jnp.logical_and(anyv, ratio >= _MIN_VOTES)
        match = ids8 == maj
        s = jnp.max(jnp.where(match, vals8, _NEG), axis=1, keepdims=True)
        scores_ref[...] = jnp.where(accepted, s, 0.0)
        classes_ref[...] = jnp.where(accepted, maj.astype(jnp.int32), -1)


@jax.jit
def _run(descriptors, places_db):
    scores, classes = pl.pallas_call(
        _body,
        grid=(_NCHUNK,),
        in_specs=[
            pl.BlockSpec((_NQ, _DIM), lambda i: (0, 0)),
            pl.BlockSpec((_CHUNK, _DIM + 1), lambda i: (i, 0)),
        ],
        out_specs=[
            pl.BlockSpec((_NQ, 1), lambda i: (0, 0)),
            pl.BlockSpec((_NQ, 1), lambda i: (0, 0)),
        ],
        out_shape=[
            jax.ShapeDtypeStruct((_NQ, 1), jnp.float32),
            jax.ShapeDtypeStruct((_NQ, 1), jnp.int32),
        ],
        scratch_shapes=[
            pltpu.VMEM((_NQ, _PAD), jnp.float32),
            pltpu.VMEM((_NQ, _PAD), jnp.float32),
            pltpu.VMEM((_NQ, _PAD), jnp.float32),
            pltpu.VMEM((_NQ, _PAD), jnp.float32),
        ],
        compiler_params=pltpu.CompilerParams(
            dimension_semantics=("arbitrary",),
        ),
    )(descriptors, places_db)
    return scores[:, 0], classes[:, 0]


def kernel(final_boxes, final_scores, final_classes, descriptors, places_db):
    scores, classes = _run(descriptors, places_db)
    return final_boxes, scores, classes


# speedup vs baseline: 2.4168x; 1.4277x over previous
import jax
import jax.numpy as jnp
from jax.experimental import pallas as pl
from jax.experimental.pallas import tpu as pltpu

_TOPK = 5
_MIN_SIM = 0.8
_MIN_VOTES = 0.0
_NQ = 64
_DIM = 64
_NDB = 100000
_CHUNK = 6400
_NCHUNK = (_NDB + _CHUNK - 1) // _CHUNK
_LANES = 128
_G = _CHUNK // _LANES
_PAD = 8
_NEG = float("-inf")


def _first_argmax(vals):
    w = vals.shape[1]
    it = jax.lax.broadcasted_iota(jnp.int32, vals.shape, 1)
    m = jnp.max(vals, axis=1, keepdims=True)
    p = jnp.min(jnp.where(vals == m, it, w), axis=1, keepdims=True)
    return m, it == p


def _body(desc_ref, db_ref, scores_ref, classes_ref, rv_ref, ri_ref):
    pid = pl.program_id(0)
    nprog = pl.num_programs(0)

    @pl.when(pid == 0)
    def _init():
        rv_ref[...] = jnp.full((_NQ, _PAD), _NEG, jnp.float32)
        ri_ref[...] = jnp.zeros((_NQ, _PAD), jnp.float32)

    rv0 = rv_ref[...]
    ri0 = ri_ref[...]

    x = db_ref[:, :_DIM]
    ids_col = db_ref[:, _DIM:_DIM + 1]
    sims = jax.lax.dot_general(
        desc_ref[...], x, (((1,), (1,)), ((), ())),
        preferred_element_type=jnp.float32)
    ids_row = ids_col.reshape(1, _CHUNK)

    t1 = jnp.full((_NQ, _LANES), _NEG, jnp.float32)
    i1 = jnp.zeros((_NQ, _LANES), jnp.float32)
    t2 = jnp.full((_NQ, _LANES), _NEG, jnp.float32)
    i2 = jnp.zeros((_NQ, _LANES), jnp.float32)
    t3 = jnp.full((_NQ, _LANES), _NEG, jnp.float32)
    for j in range(_G):
        v = sims[:, j * _LANES:(j + 1) * _LANES]
        idr = ids_row[:, j * _LANES:(j + 1) * _LANES]
        gt = v > t1
        dispv = jnp.where(gt, t1, v)
        dispi = jnp.where(gt, i1, idr)
        t1 = jnp.where(gt, v, t1)
        i1 = jnp.where(gt, idr, i1)
        gt2 = dispv > t2
        disp2 = jnp.where(gt2, t2, dispv)
        t2 = jnp.where(gt2, dispv, t2)
        i2 = jnp.where(gt2, dispi, i2)
        t3 = jnp.maximum(t3, disp2)

    all_v = jnp.concatenate([t1, t2, rv0], 1)
    all_i = jnp.concatenate([i1, i2, ri0], 1)
    pad_v = jnp.full((_NQ, _PAD - _TOPK), _NEG, jnp.float32)
    pad_z = jnp.zeros((_NQ, _PAD - _TOPK), jnp.float32)
    rvs, ris = [], []
    vals = all_v
    for _ in range(_TOPK):
        m = jnp.max(vals, axis=1, keepdims=True)
        oh = vals == m
        rvs.append(m)
        ris.append(jnp.sum(jnp.where(oh, all_i, 0.0), axis=1, keepdims=True))
        vals = jnp.where(oh, _NEG, vals)
    t5 = rvs[-1]
    bad = jnp.logical_or(jnp.any(t3 >= t5), pid == nprog - 1)
    rv_ref[...] = jnp.concatenate(rvs + [pad_v], 1)
    ri_ref[...] = jnp.concatenate(ris + [pad_z], 1)

    @pl.when(bad)
    def _exact():
        limit = _NDB - pid * _CHUNK
        it = jax.lax.broadcasted_iota(jnp.int32, (_NQ, _CHUNK), 1)
        s2 = jnp.where(it < limit, sims, _NEG)
        rit = jax.lax.broadcasted_iota(jnp.int32, (_CHUNK, 1), 0)
        idm = jnp.where(rit < limit, ids_col, 0.0)
        q_col = jnp.floor(idm * 0.125)
        r_col = idm - 8.0 * q_col
        evs, eis = [], []
        ev = s2
        for _ in range(_TOPK):
            m = jnp.max(ev, axis=1, keepdims=True)
            oh = ev == m
            ohf = oh.astype(jnp.float32)
            qsel = jax.lax.dot_general(
                ohf, q_col, (((1,), (0,)), ((), ())),
                preferred_element_type=jnp.float32)
            rsel = jax.lax.dot_general(
                ohf, r_col, (((1,), (0,)), ((), ())),
                preferred_element_type=jnp.float32)
            evs.append(m)
            eis.append(8.0 * qsel + rsel)
            ev = jnp.where(oh, _NEG, ev)
        mv = jnp.concatenate([rv0] + evs, 1)
        mi = jnp.concatenate([ri0] + eis, 1)
        nv, ni = [], []
        for _ in range(_TOPK):
            m, oh = _first_argmax(mv)
            nv.append(m)
            ni.append(jnp.sum(jnp.where(oh, mi, 0.0), axis=1, keepdims=True))
            mv = jnp.where(oh, _NEG, mv)
        rv_ref[...] = jnp.concatenate(nv + [pad_v], 1)
        ri_ref[...] = jnp.concatenate(ni + [pad_z], 1)

    @pl.when(pid == nprog - 1)
    def _finish():
        vals8 = rv_ref[...]
        ids8 = ri_ref[...]
        mask8 = vals8 >= _MIN_SIM
        counts = jnp.zeros((_NQ, _PAD), jnp.float32)
        for l in range(_TOPK):
            eq = (ids8 == ids8[:, l:l + 1]).astype(jnp.float32)
            counts = counts + jnp.where(mask8[:, l:l + 1], eq, 0.0)
        score = counts * 1e6 - ids8
        score = jnp.where(mask8, score, _NEG)
        m, oh = _first_argmax(score)
        maj = jnp.sum(jnp.where(oh, ids8, 0.0), axis=1, keepdims=True)
        majc = jnp.sum(jnp.where(oh, counts, 0.0), axis=1, keepdims=True)
        numv = jnp.sum(mask8.astype(jnp.float32), axis=1, keepdims=True)
        anyv = numv > 0
        ratio = majc / jnp.maximum(numv, 1.0)
        accepted = jnp.logical_and(anyv, ratio >= _MIN_VOTES)
        match = ids8 == maj
        s = jnp.max(jnp.where(match, vals8, _NEG), axis=1, keepdims=True)
        scores_ref[...] = jnp.where(accepted, s, 0.0)
        classes_ref[...] = jnp.where(accepted, maj.astype(jnp.int32), -1)


@jax.jit
def _run(descriptors, places_db):
    scores, classes = pl.pallas_call(
        _body,
        grid=(_NCHUNK,),
        in_specs=[
            pl.BlockSpec((_NQ, _DIM), lambda i: (0, 0)),
            pl.BlockSpec((_CHUNK, _DIM + 1), lambda i: (i, 0)),
        ],
        out_specs=[
            pl.BlockSpec((_NQ, 1), lambda i: (0, 0)),
            pl.BlockSpec((_NQ, 1), lambda i: (0, 0)),
        ],
        out_shape=[
            jax.ShapeDtypeStruct((_NQ, 1), jnp.float32),
            jax.ShapeDtypeStruct((_NQ, 1), jnp.int32),
        ],
        scratch_shapes=[
            pltpu.VMEM((_NQ, _PAD), jnp.float32),
            pltpu.VMEM((_NQ, _PAD), jnp.float32),
        ],
        compiler_params=pltpu.CompilerParams(
            dimension_semantics=("arbitrary",),
        ),
    )(descriptors, places_db)
    return scores[:, 0], classes[:, 0]


def kernel(final_boxes, final_scores, final_classes, descriptors, places_db):
    scores, classes = _run(descriptors, places_db)
    return final_boxes, scores, classes
